# baseline (speedup 1.0000x reference)
import functools

import jax
import jax.numpy as jnp
from jax import lax
from jax.experimental import pallas as pl
from jax.experimental.pallas import tpu as pltpu
from jax.experimental.pallas import tpu_sc as plsc

DIM_EMB = 32
DIM_HID = 64
DIM_OUT = 2

NUM_SC = 2
NUM_SUBCORES = 16
NW = NUM_SC * NUM_SUBCORES
CHUNK = 128
GROUP = 8
GROUP_ROWS = GROUP * CHUNK


def _make_gather(n_rows: int, d: int):
    assert n_rows % (NW * GROUP_ROWS) == 0
    rows_per_tile = n_rows // NW
    chunks_per_tile = rows_per_tile // CHUNK
    groups_per_tile = chunks_per_tile // GROUP

    mesh = plsc.VectorSubcoreMesh(core_axis_name="c", subcore_axis_name="s")

    @functools.partial(
        pl.kernel,
        mesh=mesh,
        out_type=jax.ShapeDtypeStruct((n_rows, d), jnp.float32),
        scratch_types=[
            pltpu.VMEM((chunks_per_tile, CHUNK), jnp.int32),
            pltpu.VMEM((2, GROUP_ROWS, d), jnp.float32),
            pltpu.SemaphoreType.DMA,
            pltpu.SemaphoreType.DMA,
        ],
        compiler_params=pltpu.CompilerParams(use_tc_tiling_on_sc=False),
    )
    def gather_kernel(idx_hbm, table_hbm, out_hbm, idx_v, rows_v, sem_g, sem_o):
        wid = lax.axis_index("s") * NUM_SC + lax.axis_index("c")
        chunk_base = wid * chunks_per_tile
        row_base = wid * rows_per_tile
        pltpu.sync_copy(idx_hbm.at[pl.ds(chunk_base, chunks_per_tile)], idx_v)

        def fire(g):
            buf = rows_v.at[g % 2]
            for i in range(GROUP):
                pltpu.async_copy(
                    table_hbm.at[idx_v.at[g * GROUP + i]],
                    buf.at[pl.ds(i * CHUNK, CHUNK)],
                    sem_g,
                )

        def out_slice(g):
            return out_hbm.at[pl.ds(row_base + g * GROUP_ROWS, GROUP_ROWS)]

        def drain_gathers(g):
            pltpu.make_async_copy(out_slice(g), rows_v.at[g % 2], sem_g).wait()

        def start_out(g):
            pltpu.async_copy(rows_v.at[g % 2], out_slice(g), sem_o)

        def drain_out(g):
            pltpu.make_async_copy(rows_v.at[g % 2], out_slice(g), sem_o).wait()

        fire(0)

        def body(g, carry):
            drain_gathers(g - 1)
            start_out(g - 1)

            @pl.when(g >= 2)
            def _():
                drain_out(g - 2)

            fire(g)
            return carry

        lax.fori_loop(1, groups_per_tile, body, 0)

        last = groups_per_tile - 1
        drain_gathers(last)
        start_out(last)
        drain_out(last - 1)
        drain_out(last)

    return gather_kernel


def _mlp_body(x_ref, w1p_ref, b1_ref, w2p_ref, b2_ref, o_ref):
    x3 = x_ref[...].reshape(25, 256, 128)
    acc = jnp.dot(x3[0], w1p_ref[0], preferred_element_type=jnp.float32)
    for s in range(1, 25):
        acc += jnp.dot(x3[s], w1p_ref[s], preferred_element_type=jnp.float32)
    h = jnp.maximum(acc + b1_ref[...], 0.0)
    o = jnp.dot(h, w2p_ref[...], preferred_element_type=jnp.float32) + b2_ref[...]
    oe = o[:, 0:2]
    od = o[:, 2:4]

    def lsm(z):
        m = jnp.max(z, axis=1, keepdims=True)
        s = z - m
        return s - jnp.log(jnp.sum(jnp.exp(s), axis=1, keepdims=True))

    o_ref[...] = jnp.concatenate([lsm(oe), lsm(od)], axis=1)


def kernel(inputs, embed_table, W1, b1, W2, b2):
    batch, ctx = inputs.shape
    n_rows = batch * ctx
    pairs_per_blk = 256
    slices = 2 * ctx * DIM_EMB // 128

    idxp = (
        inputs.astype(jnp.int32)
        .reshape(-1, pairs_per_blk, slices, 4)
        .transpose(0, 2, 1, 3)
        .reshape(-1, CHUNK)
    )
    emb = _make_gather(n_rows, DIM_EMB)(idxp, embed_table)
    stage = emb.reshape(n_rows * DIM_EMB // 128, 128)

    f = ctx * DIM_EMB
    z1 = jnp.zeros((f, DIM_HID), jnp.float32)
    w1p = jnp.concatenate(
        [jnp.concatenate([W1, z1], axis=1), jnp.concatenate([z1, W1], axis=1)],
        axis=0,
    ).reshape(slices, 128, 2 * DIM_HID)
    b1c = jnp.concatenate([b1, b1]).reshape(1, 2 * DIM_HID)
    z2 = jnp.zeros((DIM_HID, DIM_OUT), jnp.float32)
    w2p = jnp.concatenate(
        [jnp.concatenate([W2, z2], axis=1), jnp.concatenate([z2, W2], axis=1)],
        axis=0,
    )
    b2c = jnp.concatenate([b2, b2]).reshape(1, 2 * DIM_OUT)

    rows_per_blk = slices * pairs_per_blk
    out4 = pl.pallas_call(
        _mlp_body,
        grid=(batch // (2 * pairs_per_blk),),
        in_specs=[
            pl.BlockSpec((rows_per_blk, 128), lambda i: (i, 0)),
            pl.BlockSpec((slices, 128, 2 * DIM_HID), lambda i: (0, 0, 0)),
            pl.BlockSpec((1, 2 * DIM_HID), lambda i: (0, 0)),
            pl.BlockSpec((2 * DIM_HID, 2 * DIM_OUT), lambda i: (0, 0)),
            pl.BlockSpec((1, 2 * DIM_OUT), lambda i: (0, 0)),
        ],
        out_specs=pl.BlockSpec((pairs_per_blk, 2 * DIM_OUT), lambda i: (i, 0)),
        out_shape=jax.ShapeDtypeStruct((batch // 2, 2 * DIM_OUT), jnp.float32),
    )(stage, w1p, b1c, w2p, b2c)
    return out4.reshape(batch, DIM_OUT)


# speedup vs baseline: 2.1465x; 2.1465x over previous
import functools

import jax
import jax.numpy as jnp
from jax import lax
from jax.experimental import pallas as pl
from jax.experimental.pallas import tpu as pltpu
from jax.experimental.pallas import tpu_sc as plsc

DIM_EMB = 32
DIM_HID = 64
DIM_OUT = 2

NUM_SC = 2
NUM_SUBCORES = 16
NW = NUM_SC * NUM_SUBCORES
CHUNK = 128
GROUP = 8
GROUP_ROWS = GROUP * CHUNK
L = 16


def _make_gather(batch: int, ctx: int, d: int):
    n_rows = batch * ctx
    assert n_rows % (NW * GROUP_ROWS) == 0
    rows_per_tile = n_rows // NW
    chunks_per_tile = rows_per_tile // CHUNK
    groups_per_tile = chunks_per_tile // GROUP
    brows_per_tile = batch // NW

    mesh = plsc.VectorSubcoreMesh(core_axis_name="c", subcore_axis_name="s")

    @functools.partial(
        pl.kernel,
        mesh=mesh,
        out_type=jax.ShapeDtypeStruct((n_rows, d), jnp.float32),
        scratch_types=[
            pltpu.VMEM((brows_per_tile, ctx), jnp.int32),
            pltpu.VMEM((chunks_per_tile, CHUNK), jnp.int32),
            pltpu.VMEM((2, GROUP_ROWS, d), jnp.float32),
            pltpu.SemaphoreType.DMA,
            pltpu.SemaphoreType.DMA,
        ],
        compiler_params=pltpu.CompilerParams(use_tc_tiling_on_sc=False, needs_layout_passes=False),
    )
    def gather_kernel(idx_hbm, table_hbm, out_hbm, raw_v, idx_v, rows_v,
                      sem_g, sem_o):
        wid = lax.axis_index("s") * NUM_SC + lax.axis_index("c")
        row_base = wid * rows_per_tile
        pltpu.sync_copy(idx_hbm.at[pl.ds(wid * brows_per_tile, brows_per_tile)],
                        raw_v)

        iota = lax.iota(jnp.int32, L)

        def permute(g):
            for step in range(GROUP_ROWS // L):
                dv = jnp.full((L,), g * GROUP_ROWS + step * L, jnp.int32) + iota
                q = jnp.bitwise_and(dv, 3)
                u = jnp.bitwise_and(lax.shift_right_logical(dv, 2), 255)
                s = lax.shift_right_logical(dv, 10)
                t = s * 4 + q
                j = jnp.where(t >= ctx, 1, 0).astype(jnp.int32)
                c = t - ctx * j
                vals = plsc.load_gather(raw_v, [u * 2 + j, c])
                plsc.store_scatter(idx_v,
                                   [lax.shift_right_logical(dv, 7),
                                    jnp.bitwise_and(dv, 127)], vals)

        def fire(g):
            buf = rows_v.at[g % 2]
            for i in range(GROUP):
                pltpu.async_copy(
                    table_hbm.at[idx_v.at[g * GROUP + i]],
                    buf.at[pl.ds(i * CHUNK, CHUNK)],
                    sem_g,
                )

        def out_slice(g):
            return out_hbm.at[pl.ds(row_base + g * GROUP_ROWS, GROUP_ROWS)]

        def drain_gathers(g):
            pltpu.make_async_copy(out_slice(g), rows_v.at[g % 2], sem_g).wait()

        def start_out(g):
            pltpu.async_copy(rows_v.at[g % 2], out_slice(g), sem_o)

        def drain_out(g):
            pltpu.make_async_copy(rows_v.at[g % 2], out_slice(g), sem_o).wait()

        permute(0)
        fire(0)

        def body(g, carry):
            permute(g)
            drain_gathers(g - 1)
            start_out(g - 1)

            @pl.when(g >= 2)
            def _():
                drain_out(g - 2)

            fire(g)
            return carry

        lax.fori_loop(1, groups_per_tile, body, 0, unroll=False)

        last = groups_per_tile - 1
        drain_gathers(last)
        start_out(last)
        drain_out(last - 1)
        drain_out(last)

    return gather_kernel


def _mlp_body(x_ref, w1p_ref, b1_ref, w2p_ref, b2_ref, o_ref):
    x3 = x_ref[...].reshape(25, 256, 128)
    acc = jnp.dot(x3[0], w1p_ref[0], preferred_element_type=jnp.float32)
    for s in range(1, 25):
        acc += jnp.dot(x3[s], w1p_ref[s], preferred_element_type=jnp.float32)
    h = jnp.maximum(acc + b1_ref[...], 0.0)
    o = jnp.dot(h, w2p_ref[...], preferred_element_type=jnp.float32) + b2_ref[...]
    oe = o[:, 0:2]
    od = o[:, 2:4]

    def lsm(z):
        m = jnp.max(z, axis=1, keepdims=True)
        s = z - m
        return s - jnp.log(jnp.sum(jnp.exp(s), axis=1, keepdims=True))

    o_ref[...] = jnp.concatenate([lsm(oe), lsm(od)], axis=1)


def kernel(inputs, embed_table, W1, b1, W2, b2):
    batch, ctx = inputs.shape
    n_rows = batch * ctx
    pairs_per_blk = 256
    slices = 2 * ctx * DIM_EMB // 128

    emb = _make_gather(batch, ctx, DIM_EMB)(inputs.astype(jnp.int32), embed_table)
    stage = emb.reshape(n_rows * DIM_EMB // 128, 128)

    f = ctx * DIM_EMB
    z1 = jnp.zeros((f, DIM_HID), jnp.float32)
    w1p = jnp.concatenate(
        [jnp.concatenate([W1, z1], axis=1), jnp.concatenate([z1, W1], axis=1)],
        axis=0,
    ).reshape(slices, 128, 2 * DIM_HID)
    b1c = jnp.concatenate([b1, b1]).reshape(1, 2 * DIM_HID)
    z2 = jnp.zeros((DIM_HID, DIM_OUT), jnp.float32)
    w2p = jnp.concatenate(
        [jnp.concatenate([W2, z2], axis=1), jnp.concatenate([z2, W2], axis=1)],
        axis=0,
    )
    b2c = jnp.concatenate([b2, b2]).reshape(1, 2 * DIM_OUT)

    rows_per_blk = slices * pairs_per_blk
    out4 = pl.pallas_call(
        _mlp_body,
        grid=(batch // (2 * pairs_per_blk),),
        in_specs=[
            pl.BlockSpec((rows_per_blk, 128), lambda i: (i, 0)),
            pl.BlockSpec((slices, 128, 2 * DIM_HID), lambda i: (0, 0, 0)),
            pl.BlockSpec((1, 2 * DIM_HID), lambda i: (0, 0)),
            pl.BlockSpec((2 * DIM_HID, 2 * DIM_OUT), lambda i: (0, 0)),
            pl.BlockSpec((1, 2 * DIM_OUT), lambda i: (0, 0)),
        ],
        out_specs=pl.BlockSpec((pairs_per_blk, 2 * DIM_OUT), lambda i: (i, 0)),
        out_shape=jax.ShapeDtypeStruct((batch // 2, 2 * DIM_OUT), jnp.float32),
    )(stage, w1p, b1c, w2p, b2c)
    return out4.reshape(batch, DIM_OUT)
